# baseline (speedup 1.0000x reference)
import functools

import jax
import jax.numpy as jnp
from jax import lax
from jax.experimental import pallas as pl
from jax.experimental.pallas import tpu as pltpu
from jax.experimental.pallas import tpu_sc as plsc

VOCAB = 1000000
MAX_LEN = 200
DIM = 64
BATCH = 4096

NC = 2
NS = 16
NW = NC * NS
LANES = 16

TOTAL = BATCH * MAX_LEN
BPW = TOTAL // NW
CHUNK = 2 * MAX_LEN
NCHUNKS = BPW // CHUNK
GATHER_SPLITS = [(0, 128), (128, 128), (256, 128), (384, 16)]
assert sum(n for _, n in GATHER_SPLITS) == CHUNK


def _emb_body(x_hbm, table_hbm, pos_hbm, out_hbm,
              idx_v, pos_v, buf0, buf1, sem0, sem1):
    wid = lax.axis_index("s") * NC + lax.axis_index("c")
    base = wid * BPW

    pltpu.sync_copy(x_hbm.at[pl.ds(base, BPW)], idx_v)
    pltpu.sync_copy(pos_hbm, pos_v)

    bufs = (buf0, buf1)
    sems = (sem0, sem1)

    def start_gather(g, buf, sem):
        off = g * CHUNK
        for o, n in GATHER_SPLITS:
            pltpu.async_copy(
                table_hbm.at[idx_v.at[pl.ds(off + o, n)]],
                buf.at[pl.ds(o, n)],
                sem,
            )

    def wait_gather(buf, sem):
        pltpu.make_async_copy(table_hbm.at[pl.ds(0, CHUNK)], buf, sem).wait()

    def add_pos(buf):
        def row_body(r, _):
            for half in range(CHUNK // MAX_LEN):
                rr = half * MAX_LEN + r
                for c in range(DIM // LANES):
                    s = pl.ds(c * LANES, LANES)
                    buf[rr, s] = buf[rr, s] + pos_v[r, s]
            return _
        lax.fori_loop(0, MAX_LEN, row_body, None)

    start_gather(0, buf0, sem0)
    start_gather(1, buf1, sem1)

    def outer(og, _):
        for b in range(2):
            g = og * 2 + b
            wait_gather(bufs[b], sems[b])
            add_pos(bufs[b])
            pltpu.sync_copy(bufs[b], out_hbm.at[pl.ds(base + g * CHUNK, CHUNK)])

            @pl.when(og < NCHUNKS // 2 - 1)
            def _():
                start_gather(g + 2, bufs[b], sems[b])
        return _

    lax.fori_loop(0, NCHUNKS // 2, outer, None)


@functools.partial(jax.jit, static_argnames=())
def _emb_call(xf, class_emb, pos_emb):
    mesh = plsc.VectorSubcoreMesh(core_axis_name="c", subcore_axis_name="s")
    f = functools.partial(
        pl.kernel,
        mesh=mesh,
        out_type=jax.ShapeDtypeStruct((TOTAL, DIM), jnp.float32),
        compiler_params=pltpu.CompilerParams(use_tc_tiling_on_sc=False),
        scratch_types=[
            pltpu.VMEM((BPW,), jnp.int32),
            pltpu.VMEM((MAX_LEN, DIM), jnp.float32),
            pltpu.VMEM((CHUNK, DIM), jnp.float32),
            pltpu.VMEM((CHUNK, DIM), jnp.float32),
            pltpu.SemaphoreType.DMA,
            pltpu.SemaphoreType.DMA,
        ],
    )(_emb_body)
    return f(xf, class_emb, pos_emb)


def kernel(x, class_emb, pos_emb):
    xf = x.reshape(-1).astype(jnp.int32)
    out = _emb_call(xf, class_emb, pos_emb)
    return out.reshape(BATCH, MAX_LEN, DIM)
